# baseline (speedup 1.0000x reference)
import functools

import jax
import jax.numpy as jnp
import numpy as np
from jax import lax
from jax.experimental import pallas as pl
from jax.experimental.pallas import tpu as pltpu
from jax.experimental.pallas import tpu_sc as plsc

VOCAB = 1_000_000
D = 64
B = 4096
L = 200
BL = B * L

NC = 2
NS = 16
NW = NC * NS
PER_W = BL // NW
C = 128
CHUNKS = PER_W // C
LANES = 16
VPR = D // LANES


def _positional_encoding() -> np.ndarray:
    depth = D / 2
    positions = np.arange(L)[:, np.newaxis]
    depths = np.arange(depth)[np.newaxis, :] / depth
    angle_rates = 1 / 10000**depths
    angle_rads = positions * angle_rates
    pe = np.concatenate([np.sin(angle_rads), np.cos(angle_rads)], axis=-1)
    return pe.astype(np.float32)


_PE2 = np.concatenate([_positional_encoding()] * 2, axis=0)


def _body(table_hbm, x_hbm, pe_hbm, out_hbm, idx_v, pe_v, buf_v, sem):
    wid = lax.axis_index("s") * NC + lax.axis_index("c")
    base = wid * PER_W

    pltpu.sync_copy(x_hbm.at[pl.ds(base, PER_W)], idx_v)
    pltpu.sync_copy(pe_hbm, pe_v)

    def chunk(c, carry):
        g0 = base + c * C
        pltpu.async_copy(table_hbm.at[idx_v.at[pl.ds(c * C, C)]], buf_v,
                         sem).wait()
        pe_row0 = lax.rem(c * C, L)

        def row(r, carry2):
            pr = pe_row0 + r
            for k in range(VPR):
                sl = pl.ds(k * LANES, LANES)
                buf_v[r, sl] = buf_v[r, sl] * 8.0 + pe_v[pr, sl]
            return carry2

        lax.fori_loop(0, C, row, 0, unroll=4)
        pltpu.sync_copy(buf_v, out_hbm.at[pl.ds(g0, C)])
        return carry

    lax.fori_loop(0, CHUNKS, chunk, 0)


@jax.jit
def _run(x_flat, table, pe2):
    kern = pl.kernel(
        _body,
        out_type=jax.ShapeDtypeStruct((BL, D), jnp.float32),
        mesh=plsc.VectorSubcoreMesh(core_axis_name="c", subcore_axis_name="s"),
        compiler_params=pltpu.CompilerParams(use_tc_tiling_on_sc=False),
        scratch_types=[
            pltpu.VMEM((PER_W,), jnp.int32),
            pltpu.VMEM((2 * L, D), jnp.float32),
            pltpu.VMEM((C, D), jnp.float32),
            pltpu.SemaphoreType.DMA,
        ],
    )
    return kern(table, x_flat, pe2)


def kernel(x, table):
    x_flat = x.reshape(-1).astype(jnp.int32)
    out = _run(x_flat, table, jnp.asarray(_PE2))
    return out.reshape(B, L, D)
